# Initial kernel scaffold; baseline (speedup 1.0000x reference)
import jax
import jax.numpy as jnp
from jax.experimental import pallas as pl


def kernel(x, Wg, W1, b1, W2, b2):
    raise NotImplementedError("write your pallas kernel here")


# speedup vs baseline: 1.4989x; 1.4989x over previous
import functools

import jax
import jax.numpy as jnp
from jax import lax
from jax.experimental import pallas as pl
from jax.experimental.pallas import tpu as pltpu
from jax.experimental.pallas import tpu_sc as plsc

D = 1024
F = 4096
E = 8
T = 4096
S = T * 2
BLK = 256
G_MAX = S // BLK + (E - 1)
PADT = G_MAX * BLK
TB = 512

NSUB = 16
SLOT_PER_SUB = S // NSUB
TOK_PER_SUB = T // NSUB
ROWS_PER_W = PADT // 32


def _router_body(x_ref, wgt_ref, e0_ref, e1_ref, g0_ref, g1_ref):
    xb = x_ref[...]
    logits = jnp.dot(xb, wgt_ref[...], preferred_element_type=jnp.float32)
    eight = lax.broadcasted_iota(jnp.int32, logits.shape, 1)
    m0 = jnp.max(logits, axis=-1, keepdims=True)
    i0 = jnp.min(jnp.where(logits == m0, eight, E), axis=-1, keepdims=True)
    masked = jnp.where(eight == i0, -jnp.inf, logits)
    m1 = jnp.max(masked, axis=-1, keepdims=True)
    i1 = jnp.min(jnp.where(masked == m1, eight, E), axis=-1, keepdims=True)
    ed = jnp.exp(m1 - m0)
    g0 = 1.0 / (1.0 + ed)
    e0_ref[...] = i0
    e1_ref[...] = i1
    g0_ref[...] = g0
    g1_ref[...] = ed * g0


def _router(xf, wgT):
    outs = [jax.ShapeDtypeStruct((T, 1), jnp.int32),
            jax.ShapeDtypeStruct((T, 1), jnp.int32),
            jax.ShapeDtypeStruct((T, 1), jnp.float32),
            jax.ShapeDtypeStruct((T, 1), jnp.float32)]
    o_spec = pl.BlockSpec((TB, 1), lambda i: (i, 0))
    return pl.pallas_call(
        _router_body,
        grid=(T // TB,),
        in_specs=[pl.BlockSpec((TB, D), lambda i: (i, 0)),
                  pl.BlockSpec((D, E), lambda i: (0, 0))],
        out_specs=[o_spec, o_spec, o_spec, o_spec],
        out_shape=outs,
    )(xf, wgT)


def _dispatch_body(eidx_hbm, gate_hbm, x_hbm,
                   xs_hbm, gs_hbm, inva_hbm, invb_hbm, be_hbm,
                   kv2, ga_all, pos_c0, pos_c1, pos_c2, pos_c3,
                   base_scr, cnt_v, cc, pos_all, rid_local, gs_local,
                   inva_v, invb_v, beb, rowbuf,
                   counts_sh, pos_sh, sem):
    cid = lax.axis_index("c")
    sid = lax.axis_index("s")
    iota = lax.iota(jnp.int32, 16)
    zeros16 = jnp.zeros((16,), jnp.int32)
    s0 = sid * SLOT_PER_SUB
    pos_cs = [pos_c0, pos_c1, pos_c2, pos_c3]

    pltpu.sync_copy(eidx_hbm.at[pl.ds(sid * 4, 4)], kv2)

    def cnt_body(v, cnt):
        c = v >> 3
        l0 = (v & 7) * 16
        k = kv2[c, pl.ds(l0, 16)]
        for e in range(E):
            p = jnp.sum((k == e).astype(jnp.int32))
            cnt = jnp.where(iota == e, cnt + p, cnt)
        return cnt

    cnt_v[...] = lax.fori_loop(0, SLOT_PER_SUB // 16, cnt_body, zeros16)
    pltpu.sync_copy(cnt_v, counts_sh.at[sid])
    plsc.subcore_barrier()

    pltpu.sync_copy(counts_sh, cc)

    def tot_body(i, tot):
        return tot + cc[i]

    tot16 = lax.fori_loop(0, NSUB, tot_body, zeros16)
    pb16 = (tot16 + (BLK - 1)) >> 8
    blkoff16 = plsc.cumsum(pb16) - pb16
    poff16 = blkoff16 << 8

    def mb_body(i, b):
        take = lax.broadcast(i < sid, (16,))
        return b + jnp.where(take, cc[i], 0)

    mybase16 = lax.fori_loop(0, NSUB, mb_body, poff16)

    @pl.when((cid == 0) & (sid == 0))
    def _():
        base_scr[...] = blkoff16
        for i in range(3):
            bvec = iota + 16 * i
            be = zeros16
            for e in range(1, E):
                off_e = plsc.load_gather(
                    base_scr, [jnp.full((16,), e, jnp.int32)])
                be = be + jnp.where(bvec >= off_e, 1, 0)
            beb[pl.ds(16 * i, 16)] = be
        pltpu.sync_copy(beb, be_hbm)

    base16 = mybase16
    for c in range(4):
        def rank_body(j, b16, c=c):
            l0 = j * 16
            k = kv2[c, pl.ds(l0, 16)]
            base_scr[...] = b16
            pb = plsc.load_gather(base_scr, [k])
            rank = zeros16
            newbase = b16
            for e in range(E):
                m = k == e
                ce = plsc.cumsum(m.astype(jnp.int32))
                rank = rank + jnp.where(m, ce - 1, 0)
                newbase = jnp.where(iota == e, newbase + jnp.max(ce), newbase)
            pos_cs[c][pl.ds(l0, 16)] = pb + rank
            return newbase

        base16 = lax.fori_loop(0, 8, rank_body, base16)

    for c in range(4):
        pltpu.sync_copy(pos_cs[c], pos_sh.at[pl.ds(s0 + c * 128, 128)])

    @pl.when(cid == 0)
    def _():
        for c in range(4):
            for i in range(4):
                j16 = 2 * iota + 32 * i
                inva_v[pl.ds(64 * c + 16 * i, 16)] = plsc.load_gather(
                    pos_cs[c], [j16])
                invb_v[pl.ds(64 * c + 16 * i, 16)] = plsc.load_gather(
                    pos_cs[c], [j16 + 1])
        pltpu.sync_copy(inva_v, inva_hbm.at[pl.ds(sid * TOK_PER_SUB,
                                                  TOK_PER_SUB)])
        pltpu.sync_copy(invb_v, invb_hbm.at[pl.ds(sid * TOK_PER_SUB,
                                                  TOK_PER_SUB)])

    plsc.subcore_barrier()

    p0 = (cid * NSUB + sid) * ROWS_PER_W
    for i in range(ROWS_PER_W // 16 + 1):
        off = min(i * 16, ROWS_PER_W - 16)
        rid_local[pl.ds(p0 + off, 16)] = zeros16
        gs_local[pl.ds(p0 + off, 16)] = jnp.zeros((16,), jnp.float32)
    pltpu.sync_copy(pos_sh, pos_all)
    pltpu.sync_copy(gate_hbm, ga_all)

    def asm_body(v, carry):
        s16 = 16 * v + iota
        p16 = pos_all[pl.ds(16 * v, 16)]
        g16 = ga_all[v >> 3, pl.ds((v & 7) * 16, 16)]
        plsc.store_scatter(rid_local, [p16], lax.shift_right_logical(s16, 1))
        plsc.store_scatter(gs_local, [p16], g16)
        return carry

    lax.fori_loop(0, S // 16, asm_body, 0)

    for i in range(6):
        r0 = i * 56
        n = 56 if i < 5 else ROWS_PER_W - 5 * 56
        pltpu.async_copy(x_hbm.at[rid_local.at[pl.ds(p0 + r0, n)]],
                         rowbuf.at[pl.ds(0, n)], sem).wait()
        pltpu.sync_copy(rowbuf.at[pl.ds(0, n)],
                        xs_hbm.at[pl.ds(p0 + r0, n)])
    pltpu.sync_copy(gs_local.at[pl.ds(p0, ROWS_PER_W)],
                    gs_hbm.at[pl.ds(p0, ROWS_PER_W)])


def _dispatch(eidx3, gate3, xf):
    mesh = plsc.VectorSubcoreMesh(core_axis_name="c", subcore_axis_name="s")
    fn = pl.kernel(
        _dispatch_body,
        out_type=[jax.ShapeDtypeStruct((PADT, D), jnp.float32),
                  jax.ShapeDtypeStruct((PADT,), jnp.float32),
                  jax.ShapeDtypeStruct((T,), jnp.int32),
                  jax.ShapeDtypeStruct((T,), jnp.int32),
                  jax.ShapeDtypeStruct((48,), jnp.int32)],
        mesh=mesh,
        scratch_types=[
            pltpu.VMEM((4, 128), jnp.int32),
            pltpu.VMEM((S // 128, 128), jnp.float32),
            pltpu.VMEM((128,), jnp.int32),
            pltpu.VMEM((128,), jnp.int32),
            pltpu.VMEM((128,), jnp.int32),
            pltpu.VMEM((128,), jnp.int32),
            pltpu.VMEM((16,), jnp.int32),
            pltpu.VMEM((16,), jnp.int32),
            pltpu.VMEM((16, 16), jnp.int32),
            pltpu.VMEM((S,), jnp.int32),
            pltpu.VMEM((PADT,), jnp.int32),
            pltpu.VMEM((PADT,), jnp.float32),
            pltpu.VMEM((TOK_PER_SUB,), jnp.int32),
            pltpu.VMEM((TOK_PER_SUB,), jnp.int32),
            pltpu.VMEM((48,), jnp.int32),
            pltpu.VMEM((56, D), jnp.float32),
            pltpu.VMEM_SHARED((NSUB, 16), jnp.int32),
            pltpu.VMEM_SHARED((S,), jnp.int32),
            pltpu.SemaphoreType.DMA,
        ],
        compiler_params=pltpu.CompilerParams(needs_layout_passes=False),
    )
    return fn(eidx3, gate3, xf)


def _ffn_body(be_ref, xs_ref, w1_ref, b1_ref, w2_ref, b2_ref, gs_ref, ys_ref):
    xb = xs_ref[...].astype(jnp.bfloat16)
    h = jnp.dot(xb, w1_ref[0], preferred_element_type=jnp.float32)
    h = jax.nn.gelu(h + b1_ref[0])
    y = jnp.dot(h.astype(jnp.bfloat16), w2_ref[0],
                preferred_element_type=jnp.float32)
    ys_ref[...] = (y + b2_ref[0]) * gs_ref[0]


def _ffn(be, xs, w1b, b1, w2b, b2, gs3):
    grid_spec = pltpu.PrefetchScalarGridSpec(
        num_scalar_prefetch=1,
        grid=(G_MAX,),
        in_specs=[
            pl.BlockSpec((BLK, D), lambda g, be: (g, 0)),
            pl.BlockSpec((1, D, F), lambda g, be: (be[g], 0, 0)),
            pl.BlockSpec((1, 1, F), lambda g, be: (be[g], 0, 0)),
            pl.BlockSpec((1, F, D), lambda g, be: (be[g], 0, 0)),
            pl.BlockSpec((1, 1, D), lambda g, be: (be[g], 0, 0)),
            pl.BlockSpec((1, BLK, 1), lambda g, be: (g, 0, 0)),
        ],
        out_specs=pl.BlockSpec((BLK, D), lambda g, be: (g, 0)),
    )
    return pl.pallas_call(
        _ffn_body,
        grid_spec=grid_spec,
        out_shape=jax.ShapeDtypeStruct((PADT, D), jnp.float32),
        compiler_params=pltpu.CompilerParams(
            dimension_semantics=("arbitrary",)),
    )(be, xs, w1b, b1, w2b, b2, gs3)


def _combine_body(ys_hbm, inva_hbm, invb_hbm, out_hbm,
                  ia_v, ib_v, buf_a, buf_b, sem_a, sem_b):
    wid = lax.axis_index("c") * NSUB + lax.axis_index("s")
    t0 = wid * (T // 32)
    pltpu.sync_copy(inva_hbm.at[pl.ds(t0, T // 32)], ia_v)
    pltpu.sync_copy(invb_hbm.at[pl.ds(t0, T // 32)], ib_v)
    for i in range((T // 32) // 32):
        ca = pltpu.async_copy(ys_hbm.at[ia_v.at[pl.ds(32 * i, 32)]],
                              buf_a, sem_a)
        cb = pltpu.async_copy(ys_hbm.at[ib_v.at[pl.ds(32 * i, 32)]],
                              buf_b, sem_b)
        ca.wait()
        cb.wait()

        def add_body(r, carry):
            for c in range(D // 16):
                sl = pl.ds(c * 16, 16)
                buf_a[r, sl] = buf_a[r, sl] + buf_b[r, sl]
            return carry

        lax.fori_loop(0, 32, add_body, 0)
        pltpu.sync_copy(buf_a, out_hbm.at[pl.ds(t0 + 32 * i, 32)])


def _combine(ys, inva, invb):
    mesh = plsc.VectorSubcoreMesh(core_axis_name="c", subcore_axis_name="s")
    fn = pl.kernel(
        _combine_body,
        out_type=jax.ShapeDtypeStruct((T, D), jnp.float32),
        mesh=mesh,
        scratch_types=[
            pltpu.VMEM((T // 32,), jnp.int32),
            pltpu.VMEM((T // 32,), jnp.int32),
            pltpu.VMEM((32, D), jnp.float32),
            pltpu.VMEM((32, D), jnp.float32),
            pltpu.SemaphoreType.DMA,
            pltpu.SemaphoreType.DMA,
        ],
        compiler_params=pltpu.CompilerParams(needs_layout_passes=False),
    )
    return fn(ys, inva, invb)


def kernel(x, Wg, W1, b1, W2, b2):
    orig_shape = x.shape
    xf = x.reshape(T, D)
    e0, e1, g0, g1 = _router(xf, Wg.T)
    eidx3 = jnp.concatenate([e0, e1], axis=1).reshape(S // 128, 128)
    gate3 = jnp.concatenate([g0, g1], axis=1).reshape(S // 128, 128)
    xs, gs, inva, invb, be48 = _dispatch(eidx3, gate3, xf)
    w1b = W1.astype(jnp.bfloat16)
    w2b = W2.astype(jnp.bfloat16)
    ys = _ffn(be48[:G_MAX], xs, w1b, b1.reshape(E, 1, F),
              w2b, b2.reshape(E, 1, D), gs.reshape(G_MAX, BLK, 1))
    out = _combine(ys, inva, invb)
    return out.reshape(orig_shape)


# speedup vs baseline: 1.8807x; 1.2547x over previous
import functools

import jax
import jax.numpy as jnp
from jax import lax
from jax.experimental import pallas as pl
from jax.experimental.pallas import tpu as pltpu
from jax.experimental.pallas import tpu_sc as plsc

D = 1024
F = 4096
E = 8
T = 4096
S = T * 2
BLK = 256
G_MAX = S // BLK + (E - 1)
PADT = G_MAX * BLK
TB = 512

NSUB = 16
TOK_PER_W = T // 32


def _router_body(x_ref, wgt_ref, e0_ref, e1_ref, g0_ref, g1_ref):
    xb = x_ref[...]
    logits = jnp.dot(xb, wgt_ref[...], preferred_element_type=jnp.float32)
    eight = lax.broadcasted_iota(jnp.int32, logits.shape, 1)
    m0 = jnp.max(logits, axis=-1, keepdims=True)
    i0 = jnp.min(jnp.where(logits == m0, eight, E), axis=-1, keepdims=True)
    masked = jnp.where(eight == i0, -jnp.inf, logits)
    m1 = jnp.max(masked, axis=-1, keepdims=True)
    i1 = jnp.min(jnp.where(masked == m1, eight, E), axis=-1, keepdims=True)
    ed = jnp.exp(m1 - m0)
    g0 = 1.0 / (1.0 + ed)
    e0_ref[...] = i0
    e1_ref[...] = i1
    g0_ref[...] = g0
    g1_ref[...] = ed * g0


def _router(xf, wgT):
    outs = [jax.ShapeDtypeStruct((T, 1), jnp.int32),
            jax.ShapeDtypeStruct((T, 1), jnp.int32),
            jax.ShapeDtypeStruct((T, 1), jnp.float32),
            jax.ShapeDtypeStruct((T, 1), jnp.float32)]
    o_spec = pl.BlockSpec((TB, 1), lambda i: (i, 0))
    return pl.pallas_call(
        _router_body,
        grid=(T // TB,),
        in_specs=[pl.BlockSpec((TB, D), lambda i: (i, 0)),
                  pl.BlockSpec((D, E), lambda i: (0, 0))],
        out_specs=[o_spec, o_spec, o_spec, o_spec],
        out_shape=outs,
    )(xf, wgT)


def _dispatch_body(eidx_hbm, gate_hbm, x_hbm,
                   xs_hbm, gs2_hbm, inva_hbm, invb_hbm, be_hbm,
                   cnts_hbm,
                   kv2, kw2, gl2, pos_c0, pos_c1,
                   base_scr, cnt_v, cc, ia2, ib2,
                   inva_v, invb_v, beb, gba, gbb, bufx0, bufx1,
                   semr0, semr1, sema0, sema1, semb0, semb1, semg):
    cid = lax.axis_index("c")
    sid = lax.axis_index("s")
    w = cid * NSUB + sid
    iota = lax.iota(jnp.int32, 16)
    zeros16 = jnp.zeros((16,), jnp.int32)
    pos_cs = [pos_c0, pos_c1]

    pltpu.sync_copy(eidx_hbm.at[pl.ds(sid * 4, 4)], kv2)
    for half in range(2):
        def cnt_body(v, cnt, half=half):
            k = kv2[2 * half + (v >> 3), pl.ds((v & 7) * 16, 16)]
            for e in range(E):
                p = jnp.sum((k == e).astype(jnp.int32))
                cnt = jnp.where(iota == e, cnt + p, cnt)
            return cnt

        cnt_v[half, :] = lax.fori_loop(0, 16, cnt_body, zeros16)
    pltpu.sync_copy(cnt_v, cnts_hbm.at[cid].at[pl.ds(2 * sid, 2)])
    plsc.subcore_barrier()

    pltpu.sync_copy(cnts_hbm.at[cid], cc)

    def tot_body(i, tot):
        return tot + cc[i]

    tot16 = lax.fori_loop(0, 2 * NSUB, tot_body, zeros16)
    pb16 = (tot16 + (BLK - 1)) >> 8
    blkoff16 = plsc.cumsum(pb16) - pb16
    poff16 = blkoff16 << 8

    def mb_body(i, b):
        take = lax.broadcast(i < w, (16,))
        return b + jnp.where(take, cc[i], 0)

    mybase16 = lax.fori_loop(0, 2 * NSUB, mb_body, poff16)

    @pl.when((cid == 0) & (sid == 0))
    def _():
        base_scr[...] = blkoff16
        for i in range(3):
            bvec = iota + 16 * i
            be = zeros16
            for e in range(1, E):
                off_e = plsc.load_gather(
                    base_scr, [jnp.full((16,), e, jnp.int32)])
                be = be + jnp.where(bvec >= off_e, 1, 0)
            beb[pl.ds(16 * i, 16)] = be
        pltpu.sync_copy(beb, be_hbm)

    pltpu.sync_copy(eidx_hbm.at[pl.ds(w * 2, 2)], kw2)
    pltpu.sync_copy(gate_hbm.at[pl.ds(w * 2, 2)], gl2)
    base16 = mybase16
    for c in range(2):
        def rank_body(j, b16, c=c):
            l0 = j * 16
            k = kw2[c, pl.ds(l0, 16)]
            base_scr[...] = b16
            pb = plsc.load_gather(base_scr, [k])
            rank = zeros16
            newbase = b16
            for e in range(E):
                m = k == e
                ce = plsc.cumsum(m.astype(jnp.int32))
                rank = rank + jnp.where(m, ce - 1, 0)
                newbase = jnp.where(iota == e, newbase + jnp.max(ce), newbase)
            pos_cs[c][pl.ds(l0, 16)] = pb + rank
            return newbase

        base16 = lax.fori_loop(0, 8, rank_body, base16)

    for c in range(2):
        for i in range(4):
            j16 = 2 * iota + 32 * i
            pa = plsc.load_gather(pos_cs[c], [j16])
            pb_ = plsc.load_gather(pos_cs[c], [j16 + 1])
            inva_v[pl.ds(64 * c + 16 * i, 16)] = pa
            invb_v[pl.ds(64 * c + 16 * i, 16)] = pb_
            ia2[2 * c + (i >> 1), pl.ds((i & 1) * 16, 16)] = pa
            ib2[2 * c + (i >> 1), pl.ds((i & 1) * 16, 16)] = pb_
    pltpu.sync_copy(inva_v, inva_hbm.at[pl.ds(w * TOK_PER_W, TOK_PER_W)])
    pltpu.sync_copy(invb_v, invb_hbm.at[pl.ds(w * TOK_PER_W, TOK_PER_W)])

    for tl in range(TOK_PER_W):
        ra = jnp.full((16,), (2 * tl) >> 7, jnp.int32)
        la = jnp.full((16,), (2 * tl) & 127, jnp.int32)
        gba[tl, pl.ds(0, 16)] = plsc.load_gather(gl2, [ra, la])
        rb = jnp.full((16,), (2 * tl + 1) >> 7, jnp.int32)
        lb = jnp.full((16,), (2 * tl + 1) & 127, jnp.int32)
        gbb[tl, pl.ds(0, 16)] = plsc.load_gather(gl2, [rb, lb])

    t0 = w * TOK_PER_W
    bufs = [bufx0, bufx1]
    semr = [semr0, semr1]
    sema = [sema0, sema1]
    semb = [semb0, semb1]
    reads = [None] * 4
    scats = [None] * 4
    gscats = []
    reads[0] = pltpu.async_copy(x_hbm.at[pl.ds(t0, 32)], bufx0, semr0)
    for i in range(4):
        p = i & 1
        reads[i].wait()
        sa = pltpu.async_copy(bufs[p], xs_hbm.at[ia2.at[i]], sema[p])
        sb = pltpu.async_copy(bufs[p], xs_hbm.at[ib2.at[i]], semb[p])
        scats[i] = (sa, sb)
        gscats.append(pltpu.async_copy(gba.at[pl.ds(32 * i, 32)],
                                       gs2_hbm.at[ia2.at[i]], semg))
        gscats.append(pltpu.async_copy(gbb.at[pl.ds(32 * i, 32)],
                                       gs2_hbm.at[ib2.at[i]], semg))
        if i + 1 < 4:
            if i >= 1:
                scats[i - 1][0].wait()
                scats[i - 1][1].wait()
            reads[i + 1] = pltpu.async_copy(
                x_hbm.at[pl.ds(t0 + 32 * (i + 1), 32)], bufs[1 - p],
                semr[1 - p])
    scats[2][0].wait()
    scats[2][1].wait()
    scats[3][0].wait()
    scats[3][1].wait()
    for d in gscats:
        d.wait()


def _dispatch(eidx3, gate3, xf):
    mesh = plsc.VectorSubcoreMesh(core_axis_name="c", subcore_axis_name="s")
    fn = pl.kernel(
        _dispatch_body,
        out_type=[jax.ShapeDtypeStruct((PADT, D), jnp.float32),
                  jax.ShapeDtypeStruct((PADT, 128), jnp.float32),
                  jax.ShapeDtypeStruct((T,), jnp.int32),
                  jax.ShapeDtypeStruct((T,), jnp.int32),
                  jax.ShapeDtypeStruct((48,), jnp.int32),
                  jax.ShapeDtypeStruct((2, 2 * NSUB, 16), jnp.int32)],
        mesh=mesh,
        scratch_types=[
            pltpu.VMEM((4, 128), jnp.int32),
            pltpu.VMEM((2, 128), jnp.int32),
            pltpu.VMEM((2, 128), jnp.float32),
            pltpu.VMEM((128,), jnp.int32),
            pltpu.VMEM((128,), jnp.int32),
            pltpu.VMEM((16,), jnp.int32),
            pltpu.VMEM((2, 16), jnp.int32),
            pltpu.VMEM((2 * NSUB, 16), jnp.int32),
            pltpu.VMEM((4, 32), jnp.int32),
            pltpu.VMEM((4, 32), jnp.int32),
            pltpu.VMEM((TOK_PER_W,), jnp.int32),
            pltpu.VMEM((TOK_PER_W,), jnp.int32),
            pltpu.VMEM((48,), jnp.int32),
            pltpu.VMEM((TOK_PER_W, 128), jnp.float32),
            pltpu.VMEM((TOK_PER_W, 128), jnp.float32),
            pltpu.VMEM((32, D), jnp.float32),
            pltpu.VMEM((32, D), jnp.float32),
            pltpu.SemaphoreType.DMA,
            pltpu.SemaphoreType.DMA,
            pltpu.SemaphoreType.DMA,
            pltpu.SemaphoreType.DMA,
            pltpu.SemaphoreType.DMA,
            pltpu.SemaphoreType.DMA,
            pltpu.SemaphoreType.DMA,
        ],
        compiler_params=pltpu.CompilerParams(needs_layout_passes=False),
    )
    return fn(eidx3, gate3, xf)


def _ffn_body(be_ref, xs_ref, w1_ref, b1_ref, w2_ref, b2_ref, gs_ref, ys_ref):
    xb = xs_ref[...].astype(jnp.bfloat16)
    h = jnp.dot(xb, w1_ref[0], preferred_element_type=jnp.float32)
    h = jax.nn.gelu(h + b1_ref[0])
    y = jnp.dot(h.astype(jnp.bfloat16), w2_ref[0],
                preferred_element_type=jnp.float32)
    ys_ref[...] = (y + b2_ref[0]) * gs_ref[0][:, 0:1]


def _ffn(be, xs, w1b, b1, w2b, b2, gs3):
    grid_spec = pltpu.PrefetchScalarGridSpec(
        num_scalar_prefetch=1,
        grid=(G_MAX,),
        in_specs=[
            pl.BlockSpec((BLK, D), lambda g, be: (g, 0)),
            pl.BlockSpec((1, D, F), lambda g, be: (be[g], 0, 0)),
            pl.BlockSpec((1, 1, F), lambda g, be: (be[g], 0, 0)),
            pl.BlockSpec((1, F, D), lambda g, be: (be[g], 0, 0)),
            pl.BlockSpec((1, 1, D), lambda g, be: (be[g], 0, 0)),
            pl.BlockSpec((1, BLK, 128), lambda g, be: (g, 0, 0)),
        ],
        out_specs=pl.BlockSpec((BLK, D), lambda g, be: (g, 0)),
    )
    return pl.pallas_call(
        _ffn_body,
        grid_spec=grid_spec,
        out_shape=jax.ShapeDtypeStruct((PADT, D), jnp.float32),
        compiler_params=pltpu.CompilerParams(
            dimension_semantics=("arbitrary",)),
    )(be, xs, w1b, b1, w2b, b2, gs3)


def _combine_body(ys_hbm, inva_hbm, invb_hbm, out_hbm,
                  ia_v, ib_v, buf_a, buf_b, sem_a, sem_b):
    wid = lax.axis_index("c") * NSUB + lax.axis_index("s")
    t0 = wid * (T // 32)
    pltpu.sync_copy(inva_hbm.at[pl.ds(t0, T // 32)], ia_v)
    pltpu.sync_copy(invb_hbm.at[pl.ds(t0, T // 32)], ib_v)
    for i in range((T // 32) // 32):
        ca = pltpu.async_copy(ys_hbm.at[ia_v.at[pl.ds(32 * i, 32)]],
                              buf_a, sem_a)
        cb = pltpu.async_copy(ys_hbm.at[ib_v.at[pl.ds(32 * i, 32)]],
                              buf_b, sem_b)
        ca.wait()
        cb.wait()

        def add_body(r, carry):
            for c in range(D // 16):
                sl = pl.ds(c * 16, 16)
                buf_a[r, sl] = buf_a[r, sl] + buf_b[r, sl]
            return carry

        lax.fori_loop(0, 32, add_body, 0)
        pltpu.sync_copy(buf_a, out_hbm.at[pl.ds(t0 + 32 * i, 32)])


def _combine(ys, inva, invb):
    mesh = plsc.VectorSubcoreMesh(core_axis_name="c", subcore_axis_name="s")
    fn = pl.kernel(
        _combine_body,
        out_type=jax.ShapeDtypeStruct((T, D), jnp.float32),
        mesh=mesh,
        scratch_types=[
            pltpu.VMEM((T // 32,), jnp.int32),
            pltpu.VMEM((T // 32,), jnp.int32),
            pltpu.VMEM((32, D), jnp.float32),
            pltpu.VMEM((32, D), jnp.float32),
            pltpu.SemaphoreType.DMA,
            pltpu.SemaphoreType.DMA,
        ],
        compiler_params=pltpu.CompilerParams(needs_layout_passes=False),
    )
    return fn(ys, inva, invb)


def kernel(x, Wg, W1, b1, W2, b2):
    orig_shape = x.shape
    xf = x.reshape(T, D)
    e0, e1, g0, g1 = _router(xf, Wg.T)
    eidx3 = jnp.concatenate([e0, e1], axis=1).reshape(S // 128, 128)
    gate3 = jnp.concatenate([g0, g1], axis=1).reshape(S // 128, 128)
    xs, gs2, inva, invb, be48, _cnts = _dispatch(eidx3, gate3, xf)
    w1b = W1.astype(jnp.bfloat16)
    w2b = W2.astype(jnp.bfloat16)
    ys = _ffn(be48[:G_MAX], xs, w1b, b1.reshape(E, 1, F),
              w2b, b2.reshape(E, 1, D), gs2.reshape(G_MAX, BLK, 128))
    out = _combine(ys, inva, invb)
    return out.reshape(orig_shape)


# speedup vs baseline: 2.0572x; 1.0939x over previous
import functools

import jax
import jax.numpy as jnp
from jax import lax
from jax.experimental import pallas as pl
from jax.experimental.pallas import tpu as pltpu
from jax.experimental.pallas import tpu_sc as plsc

D = 1024
F = 4096
E = 8
T = 4096
S = T * 2
BLK = 256
G_MAX = S // BLK + (E - 1)
PADT = G_MAX * BLK
TB = 512

NSUB = 16
TOK_PER_W = T // 32


def _router_body(x_ref, wgt_ref, e0_ref, e1_ref, g0_ref, g1_ref):
    xb = x_ref[...]
    logits = jnp.dot(xb, wgt_ref[...], preferred_element_type=jnp.float32)
    eight = lax.broadcasted_iota(jnp.int32, logits.shape, 1)
    m0 = jnp.max(logits, axis=-1, keepdims=True)
    i0 = jnp.min(jnp.where(logits == m0, eight, E), axis=-1, keepdims=True)
    masked = jnp.where(eight == i0, -jnp.inf, logits)
    m1 = jnp.max(masked, axis=-1, keepdims=True)
    i1 = jnp.min(jnp.where(masked == m1, eight, E), axis=-1, keepdims=True)
    ed = jnp.exp(m1 - m0)
    g0 = 1.0 / (1.0 + ed)
    e0_ref[...] = i0
    e1_ref[...] = i1
    g0_ref[...] = g0
    g1_ref[...] = ed * g0


def _router(xf, wgT):
    outs = [jax.ShapeDtypeStruct((T, 1), jnp.int32),
            jax.ShapeDtypeStruct((T, 1), jnp.int32),
            jax.ShapeDtypeStruct((T, 1), jnp.float32),
            jax.ShapeDtypeStruct((T, 1), jnp.float32)]
    o_spec = pl.BlockSpec((TB, 1), lambda i: (i, 0))
    return pl.pallas_call(
        _router_body,
        grid=(T // TB,),
        in_specs=[pl.BlockSpec((TB, D), lambda i: (i, 0)),
                  pl.BlockSpec((D, E), lambda i: (0, 0))],
        out_specs=[o_spec, o_spec, o_spec, o_spec],
        out_shape=outs,
    )(xf, wgT)


def _dispatch_body(eidx_hbm, gate_hbm, x_hbm,
                   xs_hbm, gs2_hbm, inva_hbm, invb_hbm, be_hbm,
                   cnts_hbm,
                   kv2, kw2, gl2, pos_c0, pos_c1,
                   base_scr, cnt_v, cc, ia2, ib2,
                   inva_v, invb_v, beb, gba, gbb, bufx0, bufx1,
                   semr0, semr1, sema0, sema1, semb0, semb1, semg):
    cid = lax.axis_index("c")
    sid = lax.axis_index("s")
    w = cid * NSUB + sid
    iota = lax.iota(jnp.int32, 16)
    zeros16 = jnp.zeros((16,), jnp.int32)
    pos_cs = [pos_c0, pos_c1]

    pltpu.sync_copy(eidx_hbm.at[pl.ds(sid * 4, 4)], kv2)
    for half in range(2):
        def cnt_body(v, cnt, half=half):
            k = kv2[2 * half + (v >> 3), pl.ds((v & 7) * 16, 16)]
            for e in range(E):
                p = jnp.sum((k == e).astype(jnp.int32))
                cnt = jnp.where(iota == e, cnt + p, cnt)
            return cnt

        cnt_v[half, :] = lax.fori_loop(0, 16, cnt_body, zeros16)
    pltpu.sync_copy(cnt_v, cnts_hbm.at[cid].at[pl.ds(2 * sid, 2)])
    plsc.subcore_barrier()

    pltpu.sync_copy(cnts_hbm.at[cid], cc)

    def tot_body(i, tot):
        return tot + cc[i]

    tot16 = lax.fori_loop(0, 2 * NSUB, tot_body, zeros16)
    pb16 = (tot16 + (BLK - 1)) >> 8
    blkoff16 = plsc.cumsum(pb16) - pb16
    poff16 = blkoff16 << 8

    def mb_body(i, b):
        take = lax.broadcast(i < w, (16,))
        return b + jnp.where(take, cc[i], 0)

    mybase16 = lax.fori_loop(0, 2 * NSUB, mb_body, poff16)

    @pl.when((cid == 0) & (sid == 0))
    def _():
        base_scr[...] = blkoff16
        for i in range(3):
            bvec = iota + 16 * i
            be = zeros16
            for e in range(1, E):
                off_e = plsc.load_gather(
                    base_scr, [jnp.full((16,), e, jnp.int32)])
                be = be + jnp.where(bvec >= off_e, 1, 0)
            beb[pl.ds(16 * i, 16)] = be
        pltpu.sync_copy(beb, be_hbm)

    pltpu.sync_copy(eidx_hbm.at[pl.ds(w * 2, 2)], kw2)
    pltpu.sync_copy(gate_hbm.at[pl.ds(w * 2, 2)], gl2)
    base16 = mybase16
    for c in range(2):
        def rank_body(j, b16, c=c):
            l0 = j * 16
            k = kw2[c, pl.ds(l0, 16)]
            base_scr[...] = b16
            pb = plsc.load_gather(base_scr, [k])
            rank = zeros16
            newbase = b16
            for e in range(E):
                m = k == e
                ce = plsc.cumsum(m.astype(jnp.int32))
                rank = rank + jnp.where(m, ce - 1, 0)
                newbase = jnp.where(iota == e, newbase + jnp.max(ce), newbase)
            pos_cs[c][pl.ds(l0, 16)] = pb + rank
            return newbase

        base16 = lax.fori_loop(0, 8, rank_body, base16)

    for c in range(2):
        for i in range(4):
            j16 = 2 * iota + 32 * i
            pa = plsc.load_gather(pos_cs[c], [j16])
            pb_ = plsc.load_gather(pos_cs[c], [j16 + 1])
            inva_v[pl.ds(64 * c + 16 * i, 16)] = pa
            invb_v[pl.ds(64 * c + 16 * i, 16)] = pb_
            ia2[2 * c + (i >> 1), pl.ds((i & 1) * 16, 16)] = pa
            ib2[2 * c + (i >> 1), pl.ds((i & 1) * 16, 16)] = pb_
    pltpu.sync_copy(inva_v, inva_hbm.at[pl.ds(w * TOK_PER_W, TOK_PER_W)])
    pltpu.sync_copy(invb_v, invb_hbm.at[pl.ds(w * TOK_PER_W, TOK_PER_W)])

    for tl in range(TOK_PER_W):
        ra = jnp.full((16,), (2 * tl) >> 7, jnp.int32)
        la = jnp.full((16,), (2 * tl) & 127, jnp.int32)
        gba[tl, pl.ds(0, 16)] = plsc.load_gather(gl2, [ra, la])
        rb = jnp.full((16,), (2 * tl + 1) >> 7, jnp.int32)
        lb = jnp.full((16,), (2 * tl + 1) & 127, jnp.int32)
        gbb[tl, pl.ds(0, 16)] = plsc.load_gather(gl2, [rb, lb])

    t0 = w * TOK_PER_W
    bufs = [bufx0, bufx1]
    semr = [semr0, semr1]
    sema = [sema0, sema1]
    semb = [semb0, semb1]
    reads = [None] * 4
    scats = [None] * 4
    gscats = []
    reads[0] = pltpu.async_copy(x_hbm.at[pl.ds(t0, 32)], bufx0, semr0)
    for i in range(4):
        p = i & 1
        reads[i].wait()
        sa = pltpu.async_copy(bufs[p], xs_hbm.at[ia2.at[i]], sema[p])
        sb = pltpu.async_copy(bufs[p], xs_hbm.at[ib2.at[i]], semb[p])
        scats[i] = (sa, sb)
        gscats.append(pltpu.async_copy(gba.at[pl.ds(32 * i, 32)],
                                       gs2_hbm.at[ia2.at[i]], semg))
        gscats.append(pltpu.async_copy(gbb.at[pl.ds(32 * i, 32)],
                                       gs2_hbm.at[ib2.at[i]], semg))
        if i + 1 < 4:
            if i >= 1:
                scats[i - 1][0].wait()
                scats[i - 1][1].wait()
            reads[i + 1] = pltpu.async_copy(
                x_hbm.at[pl.ds(t0 + 32 * (i + 1), 32)], bufs[1 - p],
                semr[1 - p])
    scats[2][0].wait()
    scats[2][1].wait()
    scats[3][0].wait()
    scats[3][1].wait()
    for d in gscats:
        d.wait()


def _dispatch(eidx3, gate3, xf):
    mesh = plsc.VectorSubcoreMesh(core_axis_name="c", subcore_axis_name="s")
    fn = pl.kernel(
        _dispatch_body,
        out_type=[jax.ShapeDtypeStruct((PADT, D), jnp.float32),
                  jax.ShapeDtypeStruct((PADT, 128), jnp.float32),
                  jax.ShapeDtypeStruct((T,), jnp.int32),
                  jax.ShapeDtypeStruct((T,), jnp.int32),
                  jax.ShapeDtypeStruct((48,), jnp.int32),
                  jax.ShapeDtypeStruct((2, 2 * NSUB, 16), jnp.int32)],
        mesh=mesh,
        scratch_types=[
            pltpu.VMEM((4, 128), jnp.int32),
            pltpu.VMEM((2, 128), jnp.int32),
            pltpu.VMEM((2, 128), jnp.float32),
            pltpu.VMEM((128,), jnp.int32),
            pltpu.VMEM((128,), jnp.int32),
            pltpu.VMEM((16,), jnp.int32),
            pltpu.VMEM((2, 16), jnp.int32),
            pltpu.VMEM((2 * NSUB, 16), jnp.int32),
            pltpu.VMEM((4, 32), jnp.int32),
            pltpu.VMEM((4, 32), jnp.int32),
            pltpu.VMEM((TOK_PER_W,), jnp.int32),
            pltpu.VMEM((TOK_PER_W,), jnp.int32),
            pltpu.VMEM((48,), jnp.int32),
            pltpu.VMEM((TOK_PER_W, 128), jnp.float32),
            pltpu.VMEM((TOK_PER_W, 128), jnp.float32),
            pltpu.VMEM((32, D), jnp.float32),
            pltpu.VMEM((32, D), jnp.float32),
            pltpu.SemaphoreType.DMA,
            pltpu.SemaphoreType.DMA,
            pltpu.SemaphoreType.DMA,
            pltpu.SemaphoreType.DMA,
            pltpu.SemaphoreType.DMA,
            pltpu.SemaphoreType.DMA,
            pltpu.SemaphoreType.DMA,
        ],
        compiler_params=pltpu.CompilerParams(needs_layout_passes=False),
    )
    return fn(eidx3, gate3, xf)


def _ffn_body(be_ref, xs_ref, w1_ref, b1_ref, w2_ref, b2_ref, gs_ref, ys_ref):
    xb = xs_ref[...].astype(jnp.bfloat16)
    h = jnp.dot(xb, w1_ref[0].astype(jnp.bfloat16),
                preferred_element_type=jnp.float32)
    h = jax.nn.gelu(h + b1_ref[0])
    y = jnp.dot(h.astype(jnp.bfloat16), w2_ref[0].astype(jnp.bfloat16),
                preferred_element_type=jnp.float32)
    ys_ref[...] = (y + b2_ref[0]) * gs_ref[0][:, 0:1]


def _ffn(be, xs, w1b, b1, w2b, b2, gs3):
    grid_spec = pltpu.PrefetchScalarGridSpec(
        num_scalar_prefetch=1,
        grid=(G_MAX,),
        in_specs=[
            pl.BlockSpec((BLK, D), lambda g, be: (g, 0)),
            pl.BlockSpec((1, D, F), lambda g, be: (be[g], 0, 0),
                         pipeline_mode=pl.Buffered(buffer_count=1)),
            pl.BlockSpec((1, 1, F), lambda g, be: (be[g], 0, 0)),
            pl.BlockSpec((1, F, D), lambda g, be: (be[g], 0, 0),
                         pipeline_mode=pl.Buffered(buffer_count=1)),
            pl.BlockSpec((1, 1, D), lambda g, be: (be[g], 0, 0)),
            pl.BlockSpec((1, BLK, 128), lambda g, be: (g, 0, 0)),
        ],
        out_specs=pl.BlockSpec((BLK, D), lambda g, be: (g, 0)),
    )
    return pl.pallas_call(
        _ffn_body,
        grid_spec=grid_spec,
        out_shape=jax.ShapeDtypeStruct((PADT, D), jnp.float32),
        compiler_params=pltpu.CompilerParams(
            dimension_semantics=("arbitrary",)),
    )(be, xs, w1b, b1, w2b, b2, gs3)


def _combine_body(ys_hbm, inva_hbm, invb_hbm, out_hbm,
                  ia_v, ib_v, buf_a, buf_b, sem_a, sem_b):
    wid = lax.axis_index("c") * NSUB + lax.axis_index("s")
    t0 = wid * (T // 32)
    pltpu.sync_copy(inva_hbm.at[pl.ds(t0, T // 32)], ia_v)
    pltpu.sync_copy(invb_hbm.at[pl.ds(t0, T // 32)], ib_v)
    for i in range((T // 32) // 32):
        ca = pltpu.async_copy(ys_hbm.at[ia_v.at[pl.ds(32 * i, 32)]],
                              buf_a, sem_a)
        cb = pltpu.async_copy(ys_hbm.at[ib_v.at[pl.ds(32 * i, 32)]],
                              buf_b, sem_b)
        ca.wait()
        cb.wait()

        def add_body(r, carry):
            for c in range(D // 16):
                sl = pl.ds(c * 16, 16)
                buf_a[r, sl] = buf_a[r, sl] + buf_b[r, sl]
            return carry

        lax.fori_loop(0, 32, add_body, 0)
        pltpu.sync_copy(buf_a, out_hbm.at[pl.ds(t0 + 32 * i, 32)])


def _combine(ys, inva, invb):
    mesh = plsc.VectorSubcoreMesh(core_axis_name="c", subcore_axis_name="s")
    fn = pl.kernel(
        _combine_body,
        out_type=jax.ShapeDtypeStruct((T, D), jnp.float32),
        mesh=mesh,
        scratch_types=[
            pltpu.VMEM((T // 32,), jnp.int32),
            pltpu.VMEM((T // 32,), jnp.int32),
            pltpu.VMEM((32, D), jnp.float32),
            pltpu.VMEM((32, D), jnp.float32),
            pltpu.SemaphoreType.DMA,
            pltpu.SemaphoreType.DMA,
        ],
        compiler_params=pltpu.CompilerParams(needs_layout_passes=False),
    )
    return fn(ys, inva, invb)


def kernel(x, Wg, W1, b1, W2, b2):
    orig_shape = x.shape
    xf = x.reshape(T, D)
    e0, e1, g0, g1 = _router(xf, Wg.T)
    eidx3 = jnp.concatenate([e0, e1], axis=1).reshape(S // 128, 128)
    gate3 = jnp.concatenate([g0, g1], axis=1).reshape(S // 128, 128)
    xs, gs2, inva, invb, be48, _cnts = _dispatch(eidx3, gate3, xf)
    ys = _ffn(be48[:G_MAX], xs, W1, b1.reshape(E, 1, F),
              W2, b2.reshape(E, 1, D), gs2.reshape(G_MAX, BLK, 128))
    out = _combine(ys, inva, invb)
    return out.reshape(orig_shape)
